# baseline (speedup 1.0000x reference)
import functools
import jax
import jax.numpy as jnp
from jax import lax
from jax.experimental import pallas as pl
from jax.experimental.pallas import tpu as pltpu
from jax.experimental.pallas import tpu_sc as plsc

_NC, _NS, _L = 2, 16, 16
_NW = _NC * _NS
_H = _W = 512
_RPW = _H // _NW
_STRIP = _RPW * _W
_F32 = jnp.float32


def _sc_cnt_body(nimg, m_hbm, out_hbm, buf, sem):
    wid = lax.axis_index("s") * _NC + lax.axis_index("c")
    r0 = wid * _RPW
    zv = jnp.zeros((_L,), _F32)

    def per_image(p, _):
        pltpu.sync_copy(m_hbm.at[pl.ds((p * _H + r0) * _W, _STRIP)],
                        buf.at[pl.ds(0, _STRIP)])

        def chunk(k, acc):
            mv = buf[pl.ds(k * _L, _L)]
            return acc + jnp.where(mv >= 0.8, 1.0, 0.0)
        acc = lax.fori_loop(0, _STRIP // _L, chunk, zv)
        buf[pl.ds(_STRIP + p * _L, _L)] = acc
        return 0

    lax.fori_loop(0, nimg, per_image, 0)
    pltpu.sync_copy(buf.at[pl.ds(_STRIP, nimg * _L)],
                    out_hbm.at[pl.ds(wid * nimg * _L, nimg * _L)])


def _sc_counts(m1, nimg):
    mesh = plsc.VectorSubcoreMesh(core_axis_name="c", subcore_axis_name="s")
    return pl.kernel(
        functools.partial(_sc_cnt_body, nimg),
        out_type=jax.ShapeDtypeStruct((_NW * nimg * _L,), _F32),
        mesh=mesh,
        scratch_types=[
            pltpu.VMEM((_STRIP + nimg * _L,), _F32),
            pltpu.SemaphoreType.DMA,
        ],
    )(m1)


def _shift_rows(x, di):
    n = x.shape[0]
    r = jnp.roll(x, -di, axis=0)
    ii = jax.lax.broadcasted_iota(jnp.int32, x.shape, 0)
    if di > 0:
        return jnp.where(ii < n - di, r, 0.0)
    return jnp.where(ii >= -di, r, 0.0)


def _shift_cols(x, dj):
    n = x.shape[1]
    r = jnp.roll(x, -dj, axis=1)
    jj = jax.lax.broadcasted_iota(jnp.int32, x.shape, 1)
    if dj > 0:
        return jnp.where(jj < n - dj, r, 0.0)
    return jnp.where(jj >= -dj, r, 0.0)


def _nbsum(x):
    row = x + _shift_cols(x, 1) + _shift_cols(x, -1)
    return row + _shift_rows(row, 1) + _shift_rows(row, -1) - x


def _tc_kernel(m_ref, s_ref, out_ref):
    p = pl.program_id(0)
    m = m_ref[0]
    s = s_ref[0]
    h, w = m.shape

    ii = jax.lax.broadcasted_iota(jnp.int32, (h, w), 0)
    jj = jax.lax.broadcasted_iota(jnp.int32, (h, w), 1)
    kill = ((ii == h - 1) & (jj >= 1)) | ((jj == w - 1) & (ii >= 1))
    keep = jnp.where(kill, 0.0, 1.0)

    sum_t = _nbsum(s) * keep
    u = jnp.where(s != 0.0, sum_t, 0.0)
    v = u * jnp.abs(s - m)
    den = _nbsum(u)
    num = _nbsum(v)
    pix = (num / (den + 1.0)) * keep

    prom = m >= 0.8
    out_ref[p] = jnp.sum(jnp.where(prom, pix, 0.0))


def _tc_sums(m3, s3, nimg):
    h, w = m3.shape[1], m3.shape[2]
    return pl.pallas_call(
        _tc_kernel,
        grid=(nimg,),
        in_specs=[
            pl.BlockSpec((1, h, w), lambda p: (p, 0, 0)),
            pl.BlockSpec((1, h, w), lambda p: (p, 0, 0)),
        ],
        out_specs=pl.BlockSpec(memory_space=pltpu.SMEM),
        out_shape=jax.ShapeDtypeStruct((nimg,), _F32),
    )(m3, s3)


def kernel(out_image, segment_image, batch_num):
    b, _, h, w = out_image.shape
    m3 = out_image.reshape(b, h, w)
    s3 = segment_image.reshape(b, h, w)
    cnt_parts = _sc_counts(out_image.reshape(-1), b)
    s_sums = _tc_sums(m3, s3, b)
    cnts = cnt_parts.reshape(_NW, b, _L).sum(axis=(0, 2))
    loss = jnp.float32(0.0)
    for p in range(b):
        loss = (loss + s_sums[p]) / cnts[p]
    return loss / batch_num


# speedup vs baseline: 1.0178x; 1.0178x over previous
import functools
import jax
import jax.numpy as jnp
from jax import lax
from jax.experimental import pallas as pl
from jax.experimental.pallas import tpu as pltpu
from jax.experimental.pallas import tpu_sc as plsc

_NC, _NS, _L = 2, 16, 16
_NW = _NC * _NS
_H = _W = 512
_RPW = _H // _NW
_STRIP = _RPW * _W
_F32 = jnp.float32


_UNROLL = 8


def _sc_cnt_body(nimg, m_hbm, out_hbm, buf, sem):
    wid = lax.axis_index("s") * _NC + lax.axis_index("c")
    r0 = wid * _RPW
    zv = jnp.zeros((_L,), _F32)

    def strip_copy(p, slot):
        return pltpu.make_async_copy(
            m_hbm.at[pl.ds((p * _H + r0) * _W, _STRIP)],
            buf.at[pl.ds(slot * _STRIP, _STRIP)], sem)

    strip_copy(0, 0).start()
    for p in range(nimg):
        strip_copy(p, p % 2).wait()
        if p + 1 < nimg:
            strip_copy(p + 1, (p + 1) % 2).start()
        base = (p % 2) * _STRIP

        def chunk(k, acc, base=base):
            a0, a1 = acc
            for j in range(0, _UNROLL, 2):
                m0 = buf[pl.ds(base + (k * _UNROLL + j) * _L, _L)]
                m1 = buf[pl.ds(base + (k * _UNROLL + j + 1) * _L, _L)]
                a0 = a0 + jnp.where(m0 >= 0.8, 1.0, 0.0)
                a1 = a1 + jnp.where(m1 >= 0.8, 1.0, 0.0)
            return (a0, a1)
        a0, a1 = lax.fori_loop(0, _STRIP // (_L * _UNROLL), chunk, (zv, zv))
        buf[pl.ds(2 * _STRIP + p * _L, _L)] = a0 + a1

    pltpu.sync_copy(buf.at[pl.ds(2 * _STRIP, nimg * _L)],
                    out_hbm.at[pl.ds(wid * nimg * _L, nimg * _L)])


def _sc_counts(m1, nimg):
    mesh = plsc.VectorSubcoreMesh(core_axis_name="c", subcore_axis_name="s")
    return pl.kernel(
        functools.partial(_sc_cnt_body, nimg),
        out_type=jax.ShapeDtypeStruct((_NW * nimg * _L,), _F32),
        mesh=mesh,
        scratch_types=[
            pltpu.VMEM((2 * _STRIP + nimg * _L,), _F32),
            pltpu.SemaphoreType.DMA,
        ],
    )(m1)


def _shift_rows(x, di):
    n = x.shape[0]
    r = jnp.roll(x, -di, axis=0)
    ii = jax.lax.broadcasted_iota(jnp.int32, x.shape, 0)
    if di > 0:
        return jnp.where(ii < n - di, r, 0.0)
    return jnp.where(ii >= -di, r, 0.0)


def _shift_cols(x, dj):
    n = x.shape[1]
    r = jnp.roll(x, -dj, axis=1)
    jj = jax.lax.broadcasted_iota(jnp.int32, x.shape, 1)
    if dj > 0:
        return jnp.where(jj < n - dj, r, 0.0)
    return jnp.where(jj >= -dj, r, 0.0)


def _nbsum(x):
    row = x + _shift_cols(x, 1) + _shift_cols(x, -1)
    return row + _shift_rows(row, 1) + _shift_rows(row, -1) - x


def _tc_kernel(m_ref, s_ref, out_ref):
    p = pl.program_id(0)
    m = m_ref[0]
    s = s_ref[0]
    h, w = m.shape

    ii = jax.lax.broadcasted_iota(jnp.int32, (h, w), 0)
    jj = jax.lax.broadcasted_iota(jnp.int32, (h, w), 1)
    kill = ((ii == h - 1) & (jj >= 1)) | ((jj == w - 1) & (ii >= 1))
    keep = jnp.where(kill, 0.0, 1.0)

    sum_t = _nbsum(s) * keep
    u = jnp.where(s != 0.0, sum_t, 0.0)
    v = u * jnp.abs(s - m)
    den = _nbsum(u)
    num = _nbsum(v)
    pix = (num / (den + 1.0)) * keep

    prom = m >= 0.8
    out_ref[p] = jnp.sum(jnp.where(prom, pix, 0.0))


def _tc_sums(m3, s3, nimg):
    h, w = m3.shape[1], m3.shape[2]
    return pl.pallas_call(
        _tc_kernel,
        grid=(nimg,),
        in_specs=[
            pl.BlockSpec((1, h, w), lambda p: (p, 0, 0)),
            pl.BlockSpec((1, h, w), lambda p: (p, 0, 0)),
        ],
        out_specs=pl.BlockSpec(memory_space=pltpu.SMEM),
        out_shape=jax.ShapeDtypeStruct((nimg,), _F32),
    )(m3, s3)


def kernel(out_image, segment_image, batch_num):
    b, _, h, w = out_image.shape
    m3 = out_image.reshape(b, h, w)
    s3 = segment_image.reshape(b, h, w)
    cnt_parts = _sc_counts(out_image.reshape(-1), b)
    s_sums = _tc_sums(m3, s3, b)
    cnts = cnt_parts.reshape(_NW, b, _L).sum(axis=(0, 2))
    loss = jnp.float32(0.0)
    for p in range(b):
        loss = (loss + s_sums[p]) / cnts[p]
    return loss / batch_num


# speedup vs baseline: 1.0661x; 1.0474x over previous
import functools
import jax
import jax.numpy as jnp
from jax import lax
from jax.experimental import pallas as pl
from jax.experimental.pallas import tpu as pltpu
from jax.experimental.pallas import tpu_sc as plsc

_NC, _NS, _L = 2, 16, 16
_NW = _NC * _NS
_H = _W = 512
_NWC = _NS
_RPW = _H // _NWC
_STRIP = _RPW * _W
_F32 = jnp.float32


_UNROLL = 8


def _sc_cnt_body(nimg, m_hbm, out_hbm, buf, sem):
    wid = lax.axis_index("s") * _NC + lax.axis_index("c")
    r0 = wid * _RPW
    zv = jnp.zeros((_L,), _F32)

    def strip_copy(p, slot):
        return pltpu.make_async_copy(
            m_hbm.at[pl.ds((p * _H + r0) * _W, _STRIP)],
            buf.at[pl.ds(slot * _STRIP, _STRIP)], sem)

    strip_copy(0, 0).start()
    for p in range(nimg):
        strip_copy(p, p % 2).wait()
        if p + 1 < nimg:
            strip_copy(p + 1, (p + 1) % 2).start()
        base = (p % 2) * _STRIP

        def chunk(k, acc, base=base):
            a0, a1 = acc
            for j in range(0, _UNROLL, 2):
                m0 = buf[pl.ds(base + (k * _UNROLL + j) * _L, _L)]
                m1 = buf[pl.ds(base + (k * _UNROLL + j + 1) * _L, _L)]
                a0 = a0 + jnp.where(m0 >= 0.8, 1.0, 0.0)
                a1 = a1 + jnp.where(m1 >= 0.8, 1.0, 0.0)
            return (a0, a1)
        a0, a1 = lax.fori_loop(0, _STRIP // (_L * _UNROLL), chunk, (zv, zv))
        buf[pl.ds(2 * _STRIP + p * _L, _L)] = a0 + a1

    pltpu.sync_copy(buf.at[pl.ds(2 * _STRIP, nimg * _L)],
                    out_hbm.at[pl.ds(wid * nimg * _L, nimg * _L)])


def _sc_counts(m1, nimg):
    mesh = plsc.VectorSubcoreMesh(core_axis_name="c", subcore_axis_name="s",
                                  num_cores=1)
    return pl.kernel(
        functools.partial(_sc_cnt_body, nimg),
        out_type=jax.ShapeDtypeStruct((_NWC * nimg * _L,), _F32),
        mesh=mesh,
        scratch_types=[
            pltpu.VMEM((2 * _STRIP + nimg * _L,), _F32),
            pltpu.SemaphoreType.DMA,
        ],
    )(m1)


def _shift_rows(x, di):
    n = x.shape[0]
    r = jnp.roll(x, -di, axis=0)
    ii = jax.lax.broadcasted_iota(jnp.int32, x.shape, 0)
    if di > 0:
        return jnp.where(ii < n - di, r, 0.0)
    return jnp.where(ii >= -di, r, 0.0)


def _shift_cols(x, dj):
    n = x.shape[1]
    r = jnp.roll(x, -dj, axis=1)
    jj = jax.lax.broadcasted_iota(jnp.int32, x.shape, 1)
    if dj > 0:
        return jnp.where(jj < n - dj, r, 0.0)
    return jnp.where(jj >= -dj, r, 0.0)


def _nbsum(x):
    row = x + _shift_cols(x, 1) + _shift_cols(x, -1)
    return row + _shift_rows(row, 1) + _shift_rows(row, -1) - x


def _tc_kernel(m_ref, s_ref, out_ref):
    p = pl.program_id(0)
    m = m_ref[0]
    s = s_ref[0]
    h, w = m.shape

    ii = jax.lax.broadcasted_iota(jnp.int32, (h, w), 0)
    jj = jax.lax.broadcasted_iota(jnp.int32, (h, w), 1)
    kill = ((ii == h - 1) & (jj >= 1)) | ((jj == w - 1) & (ii >= 1))
    keep = jnp.where(kill, 0.0, 1.0)

    sum_t = _nbsum(s) * keep
    u = jnp.where(s != 0.0, sum_t, 0.0)
    v = u * jnp.abs(s - m)
    den = _nbsum(u)
    num = _nbsum(v)
    pix = (num / (den + 1.0)) * keep

    prom = m >= 0.8
    out_ref[p] = jnp.sum(jnp.where(prom, pix, 0.0))


def _tc_sums(m3, s3, nimg):
    h, w = m3.shape[1], m3.shape[2]
    return pl.pallas_call(
        _tc_kernel,
        grid=(nimg,),
        in_specs=[
            pl.BlockSpec((1, h, w), lambda p: (p, 0, 0)),
            pl.BlockSpec((1, h, w), lambda p: (p, 0, 0)),
        ],
        out_specs=pl.BlockSpec(memory_space=pltpu.SMEM),
        out_shape=jax.ShapeDtypeStruct((nimg,), _F32),
    )(m3, s3)


def kernel(out_image, segment_image, batch_num):
    b, _, h, w = out_image.shape
    m3 = out_image.reshape(b, h, w)
    s3 = segment_image.reshape(b, h, w)
    cnt_parts = _sc_counts(out_image.reshape(-1), b)
    s_sums = _tc_sums(m3, s3, b)
    cnts = cnt_parts.reshape(_NWC, b, _L).sum(axis=(0, 2))
    loss = jnp.float32(0.0)
    for p in range(b):
        loss = (loss + s_sums[p]) / cnts[p]
    return loss / batch_num


# speedup vs baseline: 2.8826x; 2.7039x over previous
import jax
import jax.numpy as jnp
from jax.experimental import pallas as pl
from jax.experimental.pallas import tpu as pltpu


def _shift_rows(x, di):
    n = x.shape[0]
    r = jnp.roll(x, -di, axis=0)
    ii = jax.lax.broadcasted_iota(jnp.int32, x.shape, 0)
    if di > 0:
        return jnp.where(ii < n - di, r, 0.0)
    return jnp.where(ii >= -di, r, 0.0)


def _shift_cols(x, dj):
    n = x.shape[1]
    r = jnp.roll(x, -dj, axis=1)
    jj = jax.lax.broadcasted_iota(jnp.int32, x.shape, 1)
    if dj > 0:
        return jnp.where(jj < n - dj, r, 0.0)
    return jnp.where(jj >= -dj, r, 0.0)


def _nbsum(x):
    row = x + _shift_cols(x, 1) + _shift_cols(x, -1)
    return row + _shift_rows(row, 1) + _shift_rows(row, -1) - x


def _loss_kernel(m_ref, s_ref, out_ref, acc_ref):
    p = pl.program_id(0)
    m = m_ref[0]
    s = s_ref[0]
    h, w = m.shape

    ii = jax.lax.broadcasted_iota(jnp.int32, (h, w), 0)
    jj = jax.lax.broadcasted_iota(jnp.int32, (h, w), 1)
    kill = ((ii == h - 1) & (jj >= 1)) | ((jj == w - 1) & (ii >= 1))

    u = jnp.where(kill | (s == 0.0), 0.0, _nbsum(s))
    v = u * jnp.abs(s - m)
    den = _nbsum(u)
    num = _nbsum(v)
    pix = jnp.where(kill, 0.0, num / (den + 1.0))

    prom = m >= 0.8
    s_sum = jnp.sum(jnp.where(prom, pix, 0.0))
    cnt = jnp.sum(prom.astype(jnp.float32))

    prev = jnp.where(p == 0, 0.0, acc_ref[0])
    new = (prev + s_sum) / cnt
    acc_ref[0] = new

    @pl.when(p == pl.num_programs(0) - 1)
    def _():
        out_ref[0] = new


def kernel(out_image, segment_image, batch_num):
    b, _, h, w = out_image.shape
    m = out_image.reshape(b, h, w)
    s = segment_image.reshape(b, h, w)
    out = pl.pallas_call(
        _loss_kernel,
        grid=(b,),
        in_specs=[
            pl.BlockSpec((1, h, w), lambda p: (p, 0, 0)),
            pl.BlockSpec((1, h, w), lambda p: (p, 0, 0)),
        ],
        out_specs=pl.BlockSpec(memory_space=pltpu.SMEM),
        out_shape=jax.ShapeDtypeStruct((1,), jnp.float32),
        scratch_shapes=[pltpu.SMEM((1,), jnp.float32)],
    )(m, s)
    return out[0] / batch_num
